# baseline (speedup 1.0000x reference)
import functools

import jax
import jax.numpy as jnp
from jax import lax
from jax.experimental import pallas as pl
from jax.experimental.pallas import tpu as pltpu
from jax.experimental.pallas import tpu_sc as plsc

_N_AMINO = 30
_N_STRUCT = 8
_D = 128
_B = 1024
_L = 200
_NW = 32
_ROWS_PER_W = _B // _NW
_NJ = _D // 16
_LPAD = _L + 16
_EPS = 1e-5


def _rsqrt(a):
    a0 = a[0]
    i = lax.bitcast_convert_type(a0, jnp.int32)
    i = jnp.int32(0x5F3759DF) - lax.shift_right_logical(i, jnp.int32(1))
    y = jnp.broadcast_to(lax.bitcast_convert_type(i, jnp.float32), (16,))
    h = a * jnp.float32(-0.5)
    for _ in range(2):
        y = y * (h * (y * y) + jnp.float32(1.5))
    return y


def _lane_sum(v, perms):
    for p in perms:
        v = v + v.at[p].get(mode="promise_in_bounds")
    return v


def _body(aseq, sseq, atab, stab, ptab, gam, bet, out,
          aidx_v, sidx_v, pos_v, comb_v, g_v, b_v, bufd, civ_v,
          sums_v, sp_v, mn_v, semd):
    wid = lax.axis_index("s") * 2 + lax.axis_index("c")
    r0 = wid * _ROWS_PER_W

    pltpu.sync_copy(aseq.at[pl.ds(r0, _ROWS_PER_W)], aidx_v)
    pltpu.sync_copy(sseq.at[pl.ds(r0, _ROWS_PER_W)], sidx_v)
    pltpu.sync_copy(ptab.at[pl.ds(0, _L)], pos_v.at[pl.ds(0, _L)])
    pltpu.sync_copy(gam, g_v)
    pltpu.sync_copy(bet, b_v)
    stage = bufd.at[0]
    pltpu.sync_copy(atab, stage.at[pl.ds(0, _N_AMINO)])
    pltpu.sync_copy(stab, stage.at[pl.ds(32, _N_STRUCT)])

    idx16 = lax.iota(jnp.int32, 16)
    perms = [idx16 ^ jnp.int32(1 << k) for k in range(4)]
    lane0 = idx16 == jnp.int32(0)
    g = [g_v[pl.ds(j * 16, 16)] for j in range(_NJ)]
    b = [b_v[pl.ds(j * 16, 16)] for j in range(_NJ)]
    inv_d = jnp.float32(1.0 / _D)

    lanemask = [idx16 == jnp.int32(k) for k in range(16)]

    def _row_sum(t):
        ts = ((t[0] + t[1]) + (t[2] + t[3])) + ((t[4] + t[5]) + (t[6] + t[7]))
        return _lane_sum(ts, perms)

    def _build(ci, carry):
        ai = lax.shift_right_logical(ci, jnp.int32(3))
        si = lax.bitwise_and(ci, jnp.int32(_N_STRUCT - 1))
        for j in range(_NJ):
            a = stage[ai, pl.ds(j * 16, 16)]
            s = stage[si + 32, pl.ds(j * 16, 16)]
            comb_v[ci, pl.ds(j * 16, 16)] = a + s
        return carry

    lax.fori_loop(jnp.int32(0), jnp.int32(_N_AMINO * _N_STRUCT), _build, 0)

    def _tsum(tg, carry):
        tb = pl.multiple_of(tg * 16, 16)
        acc = jnp.zeros((16,), jnp.float32)
        for k in range(16):
            t = [stage[tb + k, pl.ds(j * 16, 16)] for j in range(_NJ)]
            acc = jnp.where(lanemask[k], _row_sum(t), acc)
        sums_v[pl.ds(tb, 16)] = acc
        return carry

    lax.fori_loop(jnp.int32(0), jnp.int32(3), _tsum, 0)

    def _psum(pg, carry):
        pb = pl.multiple_of(pg * 16, 16)
        acc = jnp.zeros((16,), jnp.float32)
        for k in range(16):
            t = [pos_v[pb + k, pl.ds(j * 16, 16)] for j in range(_NJ)]
            acc = jnp.where(lanemask[k], _row_sum(t), acc)
        sp_v[pl.ds(pb, 16)] = acc
        return carry

    lax.fori_loop(jnp.int32(0), jnp.int32((_L + 15) // 16), _psum, 0)

    def _do_row(r, buf):
        @plsc.parallel_loop(jnp.int32(0), jnp.int32((_L + 15) // 16))
        def _pre(gi):
            gb = pl.multiple_of(gi * 16, 16)
            av = aidx_v[r, pl.ds(gb, 16)]
            sv = sidx_v[r, pl.ds(gb, 16)]
            civ = av * jnp.int32(_N_STRUCT) + sv
            civ_v[pl.ds(gb, 16)] = civ
            sa0 = sums_v[pl.ds(0, 16)]
            sa1 = sums_v[pl.ds(16, 16)]
            ssr = sums_v[pl.ds(32, 16)]
            alo = lax.bitwise_and(av, jnp.int32(15))
            sa = jnp.where(av < jnp.int32(16),
                           sa0.at[alo].get(mode="promise_in_bounds"),
                           sa1.at[alo].get(mode="promise_in_bounds"))
            ss = ssr.at[sv].get(mode="promise_in_bounds")
            mn_v[pl.ds(gb, 16)] = (sa + ss + sp_v[pl.ds(gb, 16)]) * inv_d

        @plsc.parallel_loop(jnp.int32(0), jnp.int32(_L), unroll=4)
        def _token(l):
            lbase = pl.multiple_of(lax.bitwise_and(l, jnp.int32(-16)), 16)
            k = lax.bitwise_and(l, jnp.int32(15))
            kvec = jnp.broadcast_to(k, (16,))
            civ = civ_v[pl.ds(lbase, 16)]
            rot = lax.bitwise_and(idx16 + kvec, jnp.int32(15))
            ci = civ.at[rot].get(mode="promise_in_bounds")[0]
            mrot = mn_v[pl.ds(lbase, 16)].at[rot].get(mode="promise_in_bounds")
            mean = jnp.broadcast_to(mrot[0], (16,))
            x = [comb_v[ci, pl.ds(j * 16, 16)] + pos_v[l, pl.ds(j * 16, 16)]
                 for j in range(_NJ)]
            q01 = (x[0] * x[0] + x[1] * x[1]) + (x[2] * x[2] + x[3] * x[3])
            q45 = (x[4] * x[4] + x[5] * x[5]) + (x[6] * x[6] + x[7] * x[7])
            q = _lane_sum(q01 + q45, perms)
            var = q * inv_d - mean * mean
            rstd = _rsqrt(var + jnp.float32(_EPS))
            c = jnp.float32(0.0) - mean * rstd
            for j in range(_NJ):
                y = x[j] * rstd + c
                buf[l, pl.ds(j * 16, 16)] = y * g[j] + b[j]

    buf0, buf1 = bufd.at[0], bufd.at[1]
    sem0, sem1 = semd.at[0], semd.at[1]
    _do_row(jnp.int32(0), buf0)
    pltpu.async_copy(buf0, out.at[r0], sem0)
    _do_row(jnp.int32(1), buf1)
    pltpu.async_copy(buf1, out.at[r0 + 1], sem1)

    def _pair(p, carry):
        r = p * jnp.int32(2)
        pltpu.make_async_copy(buf0, out.at[r0], sem0).wait()
        _do_row(r, buf0)
        pltpu.async_copy(buf0, out.at[r0 + r], sem0)
        pltpu.make_async_copy(buf1, out.at[r0], sem1).wait()
        _do_row(r + 1, buf1)
        pltpu.async_copy(buf1, out.at[r0 + r + 1], sem1)
        return carry

    lax.fori_loop(jnp.int32(1), jnp.int32(_ROWS_PER_W // 2), _pair, 0)
    pltpu.make_async_copy(buf0, out.at[r0], sem0).wait()
    pltpu.make_async_copy(buf1, out.at[r0], sem1).wait()


_sc_kernel = functools.partial(
    pl.kernel,
    out_type=jax.ShapeDtypeStruct((_B, _L, _D), jnp.float32),
    mesh=plsc.VectorSubcoreMesh(core_axis_name="c", subcore_axis_name="s"),
    scratch_types=[
        pltpu.VMEM((_ROWS_PER_W, _L), jnp.int32),
        pltpu.VMEM((_ROWS_PER_W, _L), jnp.int32),
        pltpu.VMEM((_LPAD // 8 * 8, _D), jnp.float32),
        pltpu.VMEM((_N_AMINO * _N_STRUCT, _D), jnp.float32),
        pltpu.VMEM((_D,), jnp.float32),
        pltpu.VMEM((_D,), jnp.float32),
        pltpu.VMEM((2, _L, _D), jnp.float32),
        pltpu.VMEM((16 * ((_L + 15) // 16),), jnp.int32),
        pltpu.VMEM((48,), jnp.float32),
        pltpu.VMEM((16 * ((_L + 15) // 16),), jnp.float32),
        pltpu.VMEM((16 * ((_L + 15) // 16),), jnp.float32),
        pltpu.SemaphoreType.DMA((2,)),
    ],
)(_body)


def kernel(amino_seq, struct_seq, amino_table, struct_table, pos_table, gamma, beta):
    return _sc_kernel(amino_seq, struct_seq, amino_table, struct_table,
                      pos_table, gamma, beta)


# speedup vs baseline: 2.2499x; 2.2499x over previous
import functools

import jax
import jax.numpy as jnp
from jax import lax
from jax.experimental import pallas as pl
from jax.experimental.pallas import tpu as pltpu
from jax.experimental.pallas import tpu_sc as plsc

_N_AMINO = 30
_N_STRUCT = 8
_D = 128
_B = 1024
_L = 200
_NW = 32
_ROWS_PER_W = _B // _NW
_NJ = _D // 16
_LPAD = _L + 16
_EPS = 1e-5


def _rsqrt(a):
    a0 = a[0]
    i = lax.bitcast_convert_type(a0, jnp.int32)
    i = jnp.int32(0x5F3759DF) - lax.shift_right_logical(i, jnp.int32(1))
    y = jnp.broadcast_to(lax.bitcast_convert_type(i, jnp.float32), (16,))
    h = a * jnp.float32(-0.5)
    for _ in range(2):
        y = y * (h * (y * y) + jnp.float32(1.5))
    return y


def _lane_sum(v, perms):
    for p in perms:
        v = v + v.at[p].get(mode="promise_in_bounds")
    return v


def _body(aseq, sseq, atab, stab, ptab, gam, bet, out,
          aidx_v, sidx_v, pos_v, comb_v, g_v, b_v, bufd, civ_v,
          sums_v, sp_v, mn_v, semd):
    wid = lax.axis_index("s") * 2 + lax.axis_index("c")
    r0 = wid * _ROWS_PER_W

    pltpu.sync_copy(aseq.at[pl.ds(r0, _ROWS_PER_W)], aidx_v)
    pltpu.sync_copy(sseq.at[pl.ds(r0, _ROWS_PER_W)], sidx_v)
    pltpu.sync_copy(ptab.at[pl.ds(0, _L)], pos_v.at[pl.ds(0, _L)])
    pltpu.sync_copy(gam, g_v)
    pltpu.sync_copy(bet, b_v)
    stage = bufd.at[0]
    pltpu.sync_copy(atab, stage.at[pl.ds(0, _N_AMINO)])
    pltpu.sync_copy(stab, stage.at[pl.ds(32, _N_STRUCT)])

    idx16 = lax.iota(jnp.int32, 16)
    perms = [idx16 ^ jnp.int32(1 << k) for k in range(4)]
    lane0 = idx16 == jnp.int32(0)
    g = [g_v[pl.ds(j * 16, 16)] for j in range(_NJ)]
    b = [b_v[pl.ds(j * 16, 16)] for j in range(_NJ)]
    inv_d = jnp.float32(1.0 / _D)

    lanemask = [idx16 == jnp.int32(k) for k in range(16)]

    def _row_sum(t):
        ts = ((t[0] + t[1]) + (t[2] + t[3])) + ((t[4] + t[5]) + (t[6] + t[7]))
        return _lane_sum(ts, perms)

    def _build(ci, carry):
        ai = lax.shift_right_logical(ci, jnp.int32(3))
        si = lax.bitwise_and(ci, jnp.int32(_N_STRUCT - 1))
        for j in range(_NJ):
            a = stage[ai, pl.ds(j * 16, 16)]
            s = stage[si + 32, pl.ds(j * 16, 16)]
            comb_v[ci, pl.ds(j * 16, 16)] = a + s
        return carry

    lax.fori_loop(jnp.int32(0), jnp.int32(_N_AMINO * _N_STRUCT), _build, 0)

    def _tsum(tg, carry):
        tb = pl.multiple_of(tg * 16, 16)
        acc = jnp.zeros((16,), jnp.float32)
        for k in range(16):
            t = [stage[tb + k, pl.ds(j * 16, 16)] for j in range(_NJ)]
            acc = jnp.where(lanemask[k], _row_sum(t), acc)
        sums_v[pl.ds(tb, 16)] = acc
        return carry

    lax.fori_loop(jnp.int32(0), jnp.int32(3), _tsum, 0)

    def _psum(pg, carry):
        pb = pl.multiple_of(pg * 16, 16)
        acc = jnp.zeros((16,), jnp.float32)
        for k in range(16):
            t = [pos_v[pb + k, pl.ds(j * 16, 16)] for j in range(_NJ)]
            acc = jnp.where(lanemask[k], _row_sum(t), acc)
        sp_v[pl.ds(pb, 16)] = acc
        return carry

    lax.fori_loop(jnp.int32(0), jnp.int32((_L + 15) // 16), _psum, 0)

    def _do_row(r, buf, apply_gb):
        @plsc.parallel_loop(jnp.int32(0), jnp.int32((_L + 15) // 16))
        def _pre(gi):
            gb = pl.multiple_of(gi * 16, 16)
            av = aidx_v[r, pl.ds(gb, 16)]
            sv = sidx_v[r, pl.ds(gb, 16)]
            civ = av * jnp.int32(_N_STRUCT) + sv
            civ_v[pl.ds(gb, 16)] = civ
            sa0 = sums_v[pl.ds(0, 16)]
            sa1 = sums_v[pl.ds(16, 16)]
            ssr = sums_v[pl.ds(32, 16)]
            alo = lax.bitwise_and(av, jnp.int32(15))
            sa = jnp.where(av < jnp.int32(16),
                           sa0.at[alo].get(mode="promise_in_bounds"),
                           sa1.at[alo].get(mode="promise_in_bounds"))
            ss = ssr.at[sv].get(mode="promise_in_bounds")
            mn_v[pl.ds(gb, 16)] = (sa + ss + sp_v[pl.ds(gb, 16)]) * inv_d

        @plsc.parallel_loop(jnp.int32(0), jnp.int32(_L), unroll=2)
        def _token(l):
            lbase = pl.multiple_of(lax.bitwise_and(l, jnp.int32(-16)), 16)
            k = lax.bitwise_and(l, jnp.int32(15))
            kvec = jnp.broadcast_to(k, (16,))
            civ = civ_v[pl.ds(lbase, 16)]
            rot = lax.bitwise_and(idx16 + kvec, jnp.int32(15))
            ci = civ.at[rot].get(mode="promise_in_bounds")[0]
            mrot = mn_v[pl.ds(lbase, 16)].at[rot].get(mode="promise_in_bounds")
            mean = jnp.broadcast_to(mrot[0], (16,))
            x = [comb_v[ci, pl.ds(j * 16, 16)] + pos_v[l, pl.ds(j * 16, 16)]
                 for j in range(_NJ)]
            q01 = (x[0] * x[0] + x[1] * x[1]) + (x[2] * x[2] + x[3] * x[3])
            q45 = (x[4] * x[4] + x[5] * x[5]) + (x[6] * x[6] + x[7] * x[7])
            q = _lane_sum(q01 + q45, perms)
            var = q * inv_d - mean * mean
            rstd = _rsqrt(var + jnp.float32(_EPS))
            c = jnp.float32(0.0) - mean * rstd
            for j in range(_NJ):
                y = x[j] * rstd + c
                if apply_gb:
                    y = y * g[j] + b[j]
                buf[l, pl.ds(j * 16, 16)] = y

    buf0, buf1 = bufd.at[0], bufd.at[1]
    sem0, sem1 = semd.at[0], semd.at[1]

    def _run_rows(apply_gb):
        _do_row(jnp.int32(0), buf0, apply_gb)
        pltpu.async_copy(buf0, out.at[r0], sem0)
        _do_row(jnp.int32(1), buf1, apply_gb)
        pltpu.async_copy(buf1, out.at[r0 + 1], sem1)

        def _pair(p, carry):
            r = p * jnp.int32(2)
            pltpu.make_async_copy(buf0, out.at[r0], sem0).wait()
            _do_row(r, buf0, apply_gb)
            pltpu.async_copy(buf0, out.at[r0 + r], sem0)
            pltpu.make_async_copy(buf1, out.at[r0], sem1).wait()
            _do_row(r + 1, buf1, apply_gb)
            pltpu.async_copy(buf1, out.at[r0 + r + 1], sem1)
            return carry

        lax.fori_loop(jnp.int32(1), jnp.int32(_ROWS_PER_W // 2), _pair, 0)
        pltpu.make_async_copy(buf0, out.at[r0], sem0).wait()
        pltpu.make_async_copy(buf1, out.at[r0], sem1).wait()

    dev = [jnp.abs(g[j] - jnp.float32(1.0)) for j in range(_NJ)]
    dev += [jnp.abs(b[j]) for j in range(_NJ)]
    m = dev[0]
    for d_ in dev[1:]:
        m = jnp.maximum(m, d_)
    for p in perms:
        m = jnp.maximum(m, m.at[p].get(mode="promise_in_bounds"))
    is_identity = m[0] == jnp.float32(0.0)

    @pl.when(is_identity)
    def _fast():
        _run_rows(False)

    @pl.when(jnp.logical_not(is_identity))
    def _general():
        _run_rows(True)


_sc_kernel = functools.partial(
    pl.kernel,
    out_type=jax.ShapeDtypeStruct((_B, _L, _D), jnp.float32),
    mesh=plsc.VectorSubcoreMesh(core_axis_name="c", subcore_axis_name="s"),
    scratch_types=[
        pltpu.VMEM((_ROWS_PER_W, _L), jnp.int32),
        pltpu.VMEM((_ROWS_PER_W, _L), jnp.int32),
        pltpu.VMEM((_LPAD // 8 * 8, _D), jnp.float32),
        pltpu.VMEM((_N_AMINO * _N_STRUCT, _D), jnp.float32),
        pltpu.VMEM((_D,), jnp.float32),
        pltpu.VMEM((_D,), jnp.float32),
        pltpu.VMEM((2, _L, _D), jnp.float32),
        pltpu.VMEM((16 * ((_L + 15) // 16),), jnp.int32),
        pltpu.VMEM((48,), jnp.float32),
        pltpu.VMEM((16 * ((_L + 15) // 16),), jnp.float32),
        pltpu.VMEM((16 * ((_L + 15) // 16),), jnp.float32),
        pltpu.SemaphoreType.DMA((2,)),
    ],
)(_body)


def kernel(amino_seq, struct_seq, amino_table, struct_table, pos_table, gamma, beta):
    return _sc_kernel(amino_seq, struct_seq, amino_table, struct_table,
                      pos_table, gamma, beta)
